# baseline (speedup 1.0000x reference)
import functools

import jax
import jax.numpy as jnp
from jax import lax
from jax.experimental import pallas as pl
from jax.experimental.pallas import tpu as pltpu
from jax.experimental.pallas import tpu_sc as plsc

_N = 10000
_E = 160000
_NC, _NS = 2, 16
_NP = 10240
_WIN = 128
_NW_AGG = 80
_NW_SRC = 88
_CH = 16
_NACC = 10112
_NW_HIST = 40
_ROWS_T = _NACC // _NS
_R = 400
_BN_S = 0.9999950000374997

_vec_mesh = plsc.VectorSubcoreMesh(core_axis_name="c", subcore_axis_name="s")


@functools.partial(
    pl.kernel,
    out_type=jax.ShapeDtypeStruct((_NC, _NP), jnp.float32),
    mesh=_vec_mesh,
    scratch_types=[
        pltpu.VMEM((_NW_HIST, _WIN), jnp.int32),
        pltpu.VMEM((_WIN,), jnp.float32),
        pltpu.VMEM((_NP // _NS,), jnp.float32),
        pltpu.VMEM_SHARED((_NP,), jnp.float32),
    ],
)
def _deg_kernel(dst_hbm, out_hbm, idx_v, ones_v, stage_v, hist_sp):
    c = lax.axis_index("c")
    s = lax.axis_index("s")
    seg = _NP // _NS

    @pl.loop(0, _WIN // 16)
    def _(i):
        ones_v[pl.ds(i * 16, 16)] = jnp.ones((16,), jnp.float32)

    @pl.loop(0, seg // 16)
    def _(i):
        stage_v[pl.ds(i * 16, 16)] = jnp.zeros((16,), jnp.float32)

    pltpu.sync_copy(stage_v, hist_sp.at[pl.ds(s * seg, seg)])
    pltpu.sync_copy(dst_hbm.at[c * _NS + s], idx_v)
    plsc.subcore_barrier()

    @pl.loop(0, _NW_HIST)
    def _(w):
        pltpu.sync_copy(ones_v, hist_sp.at[idx_v.at[w]], add=True)

    plsc.subcore_barrier()
    pltpu.sync_copy(hist_sp.at[pl.ds(s * seg, seg)], stage_v)
    pltpu.sync_copy(stage_v, out_hbm.at[c, pl.ds(s * seg, seg)])


def _make_agg(nq):
    qpc = nq // _NC

    @functools.partial(
        pl.kernel,
        out_type=jax.ShapeDtypeStruct((nq, _NP, 128), jnp.float32),
        mesh=_vec_mesh,
        scratch_types=[
            pltpu.VMEM((_NW_AGG, _WIN), jnp.int32),
            pltpu.VMEM((_NW_AGG, _WIN), jnp.int32),
            pltpu.VMEM((_WIN, 128), jnp.float32),
            pltpu.VMEM_SHARED((_NACC, 128), jnp.float32),
        ],
    )
    def agg(table_hbm, srcq_hbm, dst_hbm, out_hbm, src_v, dst_v, rows_a, acc_sp):
        c = lax.axis_index("c")
        s = lax.axis_index("s")
        pltpu.sync_copy(dst_hbm.at[s], dst_v)
        for j in range(qpc):
            q = c * qpc + j
            pltpu.sync_copy(
                table_hbm.at[pl.ds(q * _NP + s * _ROWS_T, _ROWS_T)],
                acc_sp.at[pl.ds(s * _ROWS_T, _ROWS_T)],
            )
            pltpu.sync_copy(srcq_hbm.at[q, s], src_v)
            plsc.subcore_barrier()

            @pl.loop(0, _NW_AGG)
            def _(w):
                pltpu.sync_copy(table_hbm.at[src_v.at[w]], rows_a)
                pltpu.sync_copy(rows_a, acc_sp.at[dst_v.at[w]], add=True)

            plsc.subcore_barrier()
            pltpu.sync_copy(
                acc_sp.at[pl.ds(s * _ROWS_T, _ROWS_T)],
                out_hbm.at[q, pl.ds(s * _ROWS_T, _ROWS_T)],
            )
            if j + 1 < qpc:
                plsc.subcore_barrier()

    return agg


_agg4 = _make_agg(4)
_agg2 = _make_agg(2)


def _dinv_body(h_ref, o_ref):
    d = h_ref[0, :] + h_ref[1, :] + 1.0
    o_ref[...] = lax.rsqrt(d)[:, None]


def _dinv(hist):
    return pl.pallas_call(
        _dinv_body,
        out_shape=jax.ShapeDtypeStruct((_NP, 1), jnp.float32),
    )(hist)


def _mm_first(x, w, dinv):
    no = 4

    def body(x_ref, w_ref, dv_ref, o_ref):
        p = jnp.dot(x_ref[...], w_ref[...], preferred_element_type=jnp.float32)
        o_ref[0] = p * dv_ref[...]

    return pl.pallas_call(
        body,
        grid=(_N // _R, no),
        in_specs=[
            pl.BlockSpec((_R, 256), lambda i, o: (i, 0)),
            pl.BlockSpec((256, 128), lambda i, o: (0, o)),
            pl.BlockSpec((_R, 1), lambda i, o: (i, 0)),
        ],
        out_specs=pl.BlockSpec((1, _R, 128), lambda i, o: (o, i, 0)),
        out_shape=jax.ShapeDtypeStruct((no, _NP, 128), jnp.float32),
        compiler_params=pltpu.CompilerParams(
            dimension_semantics=("parallel", "parallel"),
        ),
    )(x, w, dinv)


def _mm_mid(agg, alpha, beta, w, dinv, no):
    def body(a_ref, al_ref, be_ref, w_ref, dv_ref, o_ref):
        acc = None
        for k in range(4):
            t = a_ref[k] * dv_ref[...] * al_ref[k] + be_ref[k]
            t = jnp.maximum(t, 0.0).astype(jnp.bfloat16)
            p = jnp.dot(t, w_ref[pl.ds(k * 128, 128), :],
                        preferred_element_type=jnp.float32)
            acc = p if acc is None else acc + p
        o_ref[0] = acc * dv_ref[...]

    return pl.pallas_call(
        body,
        grid=(_N // _R, no),
        in_specs=[
            pl.BlockSpec((4, _R, 128), lambda i, o: (0, i, 0)),
            pl.BlockSpec((4, 1, 128), lambda i, o: (0, 0, 0)),
            pl.BlockSpec((4, 1, 128), lambda i, o: (0, 0, 0)),
            pl.BlockSpec((512, 128), lambda i, o: (0, o)),
            pl.BlockSpec((_R, 1), lambda i, o: (i, 0)),
        ],
        out_specs=pl.BlockSpec((1, _R, 128), lambda i, o: (o, i, 0)),
        out_shape=jax.ShapeDtypeStruct((no, _NP, 128), jnp.float32),
        compiler_params=pltpu.CompilerParams(
            dimension_semantics=("parallel", "parallel"),
        ),
    )(agg, alpha, beta, w, dinv)


def _final(agg, dinv, b3):
    def body(a_ref, dv_ref, b_ref, o_ref):
        y0 = a_ref[0] * dv_ref[...] + b_ref[0][None, :]
        y1 = a_ref[1] * dv_ref[...] + b_ref[1][None, :]
        y = jnp.concatenate([y0, y1], axis=1)
        m = jnp.max(y, axis=1, keepdims=True)
        lse = jnp.log(jnp.sum(jnp.exp(y - m), axis=1, keepdims=True)) + m
        o_ref[...] = y - lse

    return pl.pallas_call(
        body,
        grid=(_N // _R,),
        in_specs=[
            pl.BlockSpec((2, _R, 128), lambda i: (0, i, 0)),
            pl.BlockSpec((_R, 1), lambda i: (i, 0)),
            pl.BlockSpec((2, 128), lambda i: (0, 0)),
        ],
        out_specs=pl.BlockSpec((_R, 256), lambda i: (i, 0)),
        out_shape=jax.ShapeDtypeStruct((_N, 256), jnp.float32),
    )(agg, dinv, b3)


def kernel(x, adj_t, W1, b1, W2, b2, W3, b3, g1, be1, g2, be2):
    src = adj_t[0]
    dst = adj_t[1]

    pad_a = _NS * _NW_AGG * _WIN - _E
    src_p = jnp.concatenate([src, jnp.zeros((pad_a,), jnp.int32)])
    dst_p = jnp.concatenate([dst, jnp.full((pad_a,), _N, jnp.int32)])
    src_t = src_p.reshape(_NS, _NW_AGG, _WIN)
    srcq = src_t[None] + (jnp.arange(4, dtype=jnp.int32) * _NP)[:, None, None, None]
    dst_agg = dst_p.reshape(_NS, _NW_AGG, _WIN)
    pad_h = _NC * _NS * _NW_HIST * _WIN - _E
    dst_h = jnp.concatenate([dst, jnp.full((pad_h,), _N, jnp.int32)])
    dst_h = dst_h.reshape(_NC * _NS, _NW_HIST, _WIN)

    hist = _deg_kernel(dst_h)
    dinv = _dinv(hist)

    s = jnp.float32(_BN_S)
    a1 = (g1 * s).reshape(4, 1, 128)
    c1 = (b1 * g1 * s + be1).reshape(4, 1, 128)
    a2 = (g2 * s).reshape(4, 1, 128)
    c2 = (b2 * g2 * s + be2).reshape(4, 1, 128)

    h1 = _mm_first(x.astype(jnp.bfloat16), W1.astype(jnp.bfloat16), dinv)
    agg1 = _agg4(h1.reshape(4 * _NP, 128), srcq, dst_agg)
    h2 = _mm_mid(agg1, a1, c1, W2.astype(jnp.bfloat16), dinv, 4)
    agg2 = _agg4(h2.reshape(4 * _NP, 128), srcq, dst_agg)
    h3 = _mm_mid(agg2, a2, c2, W3.astype(jnp.bfloat16), dinv, 2)
    agg3 = _agg2(h3.reshape(2 * _NP, 128), srcq[:2], dst_agg)
    return _final(agg3, dinv, b3.reshape(2, 128))


# speedup vs baseline: 1.4540x; 1.4540x over previous
import functools

import jax
import jax.numpy as jnp
from jax import lax
from jax.experimental import pallas as pl
from jax.experimental.pallas import tpu as pltpu
from jax.experimental.pallas import tpu_sc as plsc

_N = 10000
_E = 160000
_NC, _NS = 2, 16
_NP = 10240
_WIN = 128
_NW_AGG = 79
_NW_SRC = 88
_CH = 16
_NACC = 10240
_NW_HIST = 40
_ROWS_T = _NACC // _NS
_R = 400
_BN_S = 0.9999950000374997

_vec_mesh = plsc.VectorSubcoreMesh(core_axis_name="c", subcore_axis_name="s")


@functools.partial(
    pl.kernel,
    out_type=jax.ShapeDtypeStruct((_NC, _NP), jnp.float32),
    mesh=_vec_mesh,
    scratch_types=[
        pltpu.VMEM((_NW_HIST, _WIN), jnp.int32),
        pltpu.VMEM((_WIN,), jnp.float32),
        pltpu.VMEM((_NP // _NS,), jnp.float32),
        pltpu.VMEM_SHARED((_NP,), jnp.float32),
    ],
)
def _deg_kernel(dst_hbm, out_hbm, idx_v, ones_v, stage_v, hist_sp):
    c = lax.axis_index("c")
    s = lax.axis_index("s")
    seg = _NP // _NS

    @pl.loop(0, _WIN // 16)
    def _(i):
        ones_v[pl.ds(i * 16, 16)] = jnp.ones((16,), jnp.float32)

    @pl.loop(0, seg // 16)
    def _(i):
        stage_v[pl.ds(i * 16, 16)] = jnp.zeros((16,), jnp.float32)

    pltpu.sync_copy(stage_v, hist_sp.at[pl.ds(s * seg, seg)])
    pltpu.sync_copy(dst_hbm.at[c * _NS + s], idx_v)
    plsc.subcore_barrier()

    @pl.loop(0, _NW_HIST)
    def _(w):
        pltpu.sync_copy(ones_v, hist_sp.at[idx_v.at[w]], add=True)

    plsc.subcore_barrier()
    pltpu.sync_copy(hist_sp.at[pl.ds(s * seg, seg)], stage_v)
    pltpu.sync_copy(stage_v, out_hbm.at[c, pl.ds(s * seg, seg)])


def _make_agg(nq):
    qpc = nq // _NC

    @functools.partial(
        pl.kernel,
        out_type=jax.ShapeDtypeStruct((nq, _NP, 128), jnp.float32),
        mesh=_vec_mesh,
        scratch_types=[
            pltpu.VMEM((_NW_AGG, _WIN), jnp.int32),
            pltpu.VMEM((_NW_AGG, _WIN), jnp.int32),
            pltpu.VMEM((_WIN, 128), jnp.float32),
            pltpu.VMEM_SHARED((_NACC, 128), jnp.float32),
        ],
    )
    def agg(table_hbm, srcq_hbm, dst_hbm, out_hbm, src_v, dst_v, rows_a, acc_sp):
        c = lax.axis_index("c")
        s = lax.axis_index("s")
        pltpu.sync_copy(dst_hbm.at[s], dst_v)
        for j in range(qpc):
            q = c * qpc + j
            pltpu.sync_copy(
                table_hbm.at[pl.ds(q * _NP + s * _ROWS_T, _ROWS_T)],
                acc_sp.at[pl.ds(s * _ROWS_T, _ROWS_T)],
            )
            pltpu.sync_copy(srcq_hbm.at[q, s], src_v)
            plsc.subcore_barrier()

            @pl.loop(0, _NW_AGG)
            def _(w):
                pltpu.sync_copy(table_hbm.at[src_v.at[w]], rows_a)
                pltpu.sync_copy(rows_a, acc_sp.at[dst_v.at[w]], add=True)

            plsc.subcore_barrier()
            pltpu.sync_copy(
                acc_sp.at[pl.ds(s * _ROWS_T, _ROWS_T)],
                out_hbm.at[q, pl.ds(s * _ROWS_T, _ROWS_T)],
            )
            if j + 1 < qpc:
                plsc.subcore_barrier()

    return agg


_agg4 = _make_agg(4)
_agg2 = _make_agg(2)


def _dinv_body(h_ref, o_ref):
    d = h_ref[0, :] + h_ref[1, :] + 1.0
    o_ref[...] = lax.rsqrt(d)[:, None]


def _dinv(hist):
    return pl.pallas_call(
        _dinv_body,
        out_shape=jax.ShapeDtypeStruct((_NP, 1), jnp.float32),
    )(hist)


def _mm_first(x, w, dinv):
    no = 4

    def body(x_ref, w_ref, dv_ref, o_ref):
        p = jnp.dot(x_ref[...], w_ref[...], preferred_element_type=jnp.float32)
        o_ref[0] = p * dv_ref[...]

    return pl.pallas_call(
        body,
        grid=(_N // _R, no),
        in_specs=[
            pl.BlockSpec((_R, 256), lambda i, o: (i, 0)),
            pl.BlockSpec((256, 128), lambda i, o: (0, o)),
            pl.BlockSpec((_R, 1), lambda i, o: (i, 0)),
        ],
        out_specs=pl.BlockSpec((1, _R, 128), lambda i, o: (o, i, 0)),
        out_shape=jax.ShapeDtypeStruct((no, _NP, 128), jnp.float32),
        compiler_params=pltpu.CompilerParams(
            dimension_semantics=("parallel", "parallel"),
        ),
    )(x, w, dinv)


def _mm_mid(agg, alpha, beta, w, dinv, no):
    def body(a_ref, al_ref, be_ref, w_ref, dv_ref, o_ref):
        acc = None
        for k in range(4):
            t = a_ref[k] * dv_ref[...] * al_ref[k] + be_ref[k]
            t = jnp.maximum(t, 0.0).astype(jnp.bfloat16)
            p = jnp.dot(t, w_ref[pl.ds(k * 128, 128), :],
                        preferred_element_type=jnp.float32)
            acc = p if acc is None else acc + p
        o_ref[0] = acc * dv_ref[...]

    return pl.pallas_call(
        body,
        grid=(_N // _R, no),
        in_specs=[
            pl.BlockSpec((4, _R, 128), lambda i, o: (0, i, 0)),
            pl.BlockSpec((4, 1, 128), lambda i, o: (0, 0, 0)),
            pl.BlockSpec((4, 1, 128), lambda i, o: (0, 0, 0)),
            pl.BlockSpec((512, 128), lambda i, o: (0, o)),
            pl.BlockSpec((_R, 1), lambda i, o: (i, 0)),
        ],
        out_specs=pl.BlockSpec((1, _R, 128), lambda i, o: (o, i, 0)),
        out_shape=jax.ShapeDtypeStruct((no, _NP, 128), jnp.float32),
        compiler_params=pltpu.CompilerParams(
            dimension_semantics=("parallel", "parallel"),
        ),
    )(agg, alpha, beta, w, dinv)


def _final(agg, dinv, b3):
    def body(a_ref, dv_ref, b_ref, o_ref):
        y0 = a_ref[0] * dv_ref[...] + b_ref[0][None, :]
        y1 = a_ref[1] * dv_ref[...] + b_ref[1][None, :]
        y = jnp.concatenate([y0, y1], axis=1)
        m = jnp.max(y, axis=1, keepdims=True)
        lse = jnp.log(jnp.sum(jnp.exp(y - m), axis=1, keepdims=True)) + m
        o_ref[...] = y - lse

    return pl.pallas_call(
        body,
        grid=(_N // _R,),
        in_specs=[
            pl.BlockSpec((2, _R, 128), lambda i: (0, i, 0)),
            pl.BlockSpec((_R, 1), lambda i: (i, 0)),
            pl.BlockSpec((2, 128), lambda i: (0, 0)),
        ],
        out_specs=pl.BlockSpec((_R, 256), lambda i: (i, 0)),
        out_shape=jax.ShapeDtypeStruct((_N, 256), jnp.float32),
    )(agg, dinv, b3)


def kernel(x, adj_t, W1, b1, W2, b2, W3, b3, g1, be1, g2, be2):
    src = adj_t[0]
    dst = adj_t[1]

    pad_a = _NS * _NW_AGG * _WIN - _E
    src_p = jnp.concatenate([src, jnp.zeros((pad_a,), jnp.int32)])
    dst_p = jnp.concatenate([dst, jnp.full((pad_a,), _N, jnp.int32)])
    src_t = src_p.reshape(_NS, _NW_AGG, _WIN)
    srcq = src_t[None] + (jnp.arange(4, dtype=jnp.int32) * _NP)[:, None, None, None]
    dst_agg = dst_p.reshape(_NS, _NW_AGG, _WIN)
    pad_h = _NC * _NS * _NW_HIST * _WIN - _E
    dst_h = jnp.concatenate([dst, jnp.full((pad_h,), _N, jnp.int32)])
    dst_h = dst_h.reshape(_NC * _NS, _NW_HIST, _WIN)

    hist = _deg_kernel(dst_h)
    dinv = _dinv(hist)

    s = jnp.float32(_BN_S)
    a1 = (g1 * s).reshape(4, 1, 128)
    c1 = (b1 * g1 * s + be1).reshape(4, 1, 128)
    a2 = (g2 * s).reshape(4, 1, 128)
    c2 = (b2 * g2 * s + be2).reshape(4, 1, 128)

    h1 = _mm_first(x.astype(jnp.bfloat16), W1.astype(jnp.bfloat16), dinv)
    agg1 = _agg4(h1.reshape(4 * _NP, 128), srcq, dst_agg)
    h2 = _mm_mid(agg1, a1, c1, W2.astype(jnp.bfloat16), dinv, 4)
    agg2 = _agg4(h2.reshape(4 * _NP, 128), srcq, dst_agg)
    h3 = _mm_mid(agg2, a2, c2, W3.astype(jnp.bfloat16), dinv, 2)
    agg3 = _agg2(h3.reshape(2 * _NP, 128), srcq[:2], dst_agg)
    return _final(agg3, dinv, b3.reshape(2, 128))
